# baseline (speedup 1.0000x reference)
import functools

import jax
import jax.numpy as jnp
from jax import lax
from jax.experimental import pallas as pl
from jax.experimental.pallas import tpu as pltpu
from jax.experimental.pallas import tpu_sc as plsc

_NI_TOT = 16384
_NJ = 26
_B_ROWS = _NI_TOT * _NJ
_DIM = 64
_NC, _NS = 2, 16
_NW = _NC * _NS
_BPW = _B_ROWS // _NW
_NI = 8
_CH = _NI * _NJ
_NCH = _BPW // _CH
_IPW = _NI_TOT // _NW
_NBUF = 2
_XPAD = 128
_NJ8 = 32

_mesh = plsc.VectorSubcoreMesh(core_axis_name="c", subcore_axis_name="s")


@functools.partial(
    pl.kernel,
    out_type=jax.ShapeDtypeStruct((_NI_TOT, _NJ, _DIM), jnp.float32),
    mesh=_mesh,
    scratch_types=[
        pltpu.VMEM((_IPW, _XPAD), jnp.int32),
        pltpu.VMEM((_NBUF, _NI * _NJ8, _DIM), jnp.float32),
        [pltpu.SemaphoreType.DMA] * _NBUF,
        [pltpu.SemaphoreType.DMA] * _NBUF,
    ],
    compiler_params=pltpu.CompilerParams(use_tc_tiling_on_sc=False),
)
def _sc_gather(idx_hbm, table_hbm, out_hbm, idx_v, rows_v, gsems, wsems):
    wid = lax.axis_index("s") * _NC + lax.axis_index("c")
    i_base = wid * _IPW

    pltpu.sync_copy(idx_hbm.at[pl.ds(i_base, _IPW)], idx_v)

    def _start_gather(g, b):
        for di in range(_NI):
            pltpu.async_copy(
                table_hbm.at[idx_v.at[g * _NI + di, pl.ds(0, _NJ8)]],
                rows_v.at[b, pl.ds(di * _NJ8, _NJ8)],
                gsems[b],
            )

    def _wait_gather(b):
        for di in range(_NI):
            pltpu.make_async_copy(
                table_hbm.at[idx_v.at[0, pl.ds(0, _NJ8)]],
                rows_v.at[b, pl.ds(di * _NJ8, _NJ8)],
                gsems[b],
            ).wait()

    def _start_writes(g, b):
        for di in range(_NI):
            pltpu.async_copy(
                rows_v.at[b, pl.ds(di * _NJ8, _NJ)],
                out_hbm.at[i_base + g * _NI + di],
                wsems[b],
            )

    def _wait_writes(g, b):
        for di in range(_NI):
            pltpu.make_async_copy(
                rows_v.at[b, pl.ds(di * _NJ8, _NJ)],
                out_hbm.at[i_base + g * _NI + di],
                wsems[b],
            ).wait()

    for b in range(_NBUF):
        _start_gather(b, b)

    @pl.loop(0, _NCH - _NBUF, step=_NBUF)
    def _body(jj):
        for b in range(_NBUF):
            _wait_gather(b)
            _start_writes(jj + b, b)
        for b in range(_NBUF):
            _wait_writes(jj + b, b)
            _start_gather(jj + b + _NBUF, b)

    for b in range(_NBUF):
        g = _NCH - _NBUF + b
        _wait_gather(b)
        _start_writes(g, b)
        _wait_writes(g, b)


def kernel(x, weight):
    xp = jnp.pad(x.astype(jnp.int32), ((0, 0), (0, _XPAD - _NJ)))
    return _sc_gather(xp, weight)


# speedup vs baseline: 3.8272x; 3.8272x over previous
import functools

import jax
import jax.numpy as jnp
from jax import lax
from jax.experimental import pallas as pl
from jax.experimental.pallas import tpu as pltpu
from jax.experimental.pallas import tpu_sc as plsc

_NI_TOT = 16384
_NJ = 26
_B_ROWS = _NI_TOT * _NJ
_DIM = 64
_NC, _NS = 2, 16
_NW = _NC * _NS
_BPW = _B_ROWS // _NW
_IPW = _NI_TOT // _NW
_NI = 8
_CH = _NI * _NJ
_NCH = _BPW // _CH
_NBUF = 4

_mesh = plsc.VectorSubcoreMesh(core_axis_name="c", subcore_axis_name="s")


@functools.partial(
    pl.kernel,
    out_type=jax.ShapeDtypeStruct((_NI_TOT, 32, 2 * _DIM), jnp.float32),
    mesh=_mesh,
    scratch_types=[
        pltpu.VMEM((_BPW,), jnp.int32),
        pltpu.VMEM((_NBUF, _CH, _DIM), jnp.float32),
        [pltpu.SemaphoreType.DMA] * _NBUF,
        [pltpu.SemaphoreType.DMA] * _NBUF,
    ],
    compiler_params=pltpu.CompilerParams(use_tc_tiling_on_sc=False),
)
def _sc_gather(idx_hbm, table_hbm, out_hbm, idx_v, rows_v, gsems, wsems):
    wid = lax.axis_index("s") * _NC + lax.axis_index("c")
    i_base = wid * _IPW

    pltpu.sync_copy(idx_hbm.at[pl.ds(wid * _BPW, _BPW)], idx_v)

    def _start_gather(g, b):
        pltpu.async_copy(
            table_hbm.at[idx_v.at[pl.ds(g * _CH, _CH)]], rows_v.at[b], gsems[b]
        )

    def _wait_gather(b):
        pltpu.make_async_copy(
            table_hbm.at[idx_v.at[pl.ds(0, _CH)]], rows_v.at[b], gsems[b]
        ).wait()

    def _start_writes(g, b):
        for di in range(_NI):
            pltpu.async_copy(
                rows_v.at[b, pl.ds(di * _NJ, _NJ)],
                out_hbm.at[i_base + g * _NI + di, pl.ds(0, _NJ), pl.ds(0, _DIM)],
                wsems[b],
            )

    def _wait_writes(g, b):
        for di in range(_NI):
            pltpu.make_async_copy(
                rows_v.at[b, pl.ds(di * _NJ, _NJ)],
                out_hbm.at[i_base + g * _NI + di, pl.ds(0, _NJ), pl.ds(0, _DIM)],
                wsems[b],
            ).wait()

    for b in range(_NBUF):
        _start_gather(b, b)

    @pl.loop(0, _NCH - _NBUF, step=_NBUF)
    def _body(jj):
        for b in range(_NBUF):
            _wait_gather(b)
            _start_writes(jj + b, b)
        for b in range(_NBUF):
            _wait_writes(jj + b, b)
            _start_gather(jj + b + _NBUF, b)

    for b in range(_NBUF):
        g = _NCH - _NBUF + b
        _wait_gather(b)
        _start_writes(g, b)
        _wait_writes(g, b)


def kernel(x, weight):
    idx = x.astype(jnp.int32).reshape(_B_ROWS)
    inter = _sc_gather(idx, weight)
    return inter[:, :_NJ, :_DIM]
